# baseline (speedup 1.0000x reference)
import jax
import jax.numpy as jnp
from jax import lax
from jax.experimental import pallas as pl
from jax.experimental.pallas import tpu as pltpu
from jax.experimental.pallas import tpu_sc as plsc

_N_ENTRIES = 262144
_H = 8192
_T = 131072
_J = 15
_W = 32
_HPW = _H // _W
_NBH = 8
_SPH = 128 * _J
_CH = 128


def _hint_kernel(entries_hbm, blk_hbm, off_hbm, starts_hbm, sizes_hbm, bs_hbm,
                 out_hbm, starts_v, sizes_v, rowidx_v, bwin_v, owin_v, eilist_v,
                 lowlist_v, rows_v, outbuf_v, bs_v, sem_a, sem_b):
    wid = lax.axis_index("s") * 2 + lax.axis_index("c")
    base = wid * _HPW
    iota = lax.iota(jnp.int32, 16)

    pltpu.sync_copy(starts_hbm.at[pl.ds(base, _HPW)], starts_v)
    pltpu.sync_copy(sizes_hbm.at[pl.ds(base, _HPW)], sizes_v)
    pltpu.sync_copy(bs_hbm, bs_v)

    def stage1(b, c):
        s = starts_v[pl.ds(16 * b, 16)]
        r = jnp.right_shift(s, 4)
        rowidx_v[pl.ds(16 * b, 16)] = r
        rowidx_v[pl.ds(_HPW + 16 * b, 16)] = r + 1
        return c

    lax.fori_loop(jnp.int32(0), jnp.int32(16), stage1, 0)

    descs = []
    for c in range(2 * _HPW // _CH):
        descs.append(pltpu.async_copy(
            blk_hbm.at[rowidx_v.at[pl.ds(_CH * c, _CH)]],
            bwin_v.at[pl.ds(_CH * c, _CH)], sem_a))
        descs.append(pltpu.async_copy(
            off_hbm.at[rowidx_v.at[pl.ds(_CH * c, _CH)]],
            owin_v.at[pl.ds(_CH * c, _CH)], sem_a))
    for d in descs:
        d.wait()

    for h in range(2):
        def stage3(b, c, h=h):
            i0 = 128 * h + 16 * b
            s = starts_v[pl.ds(i0, 16)]
            w0 = jnp.bitwise_and(s, 15)
            i_vec = iota + i0
            bs = bs_v[...]
            for j in range(_J):
                w = w0 + j
                row = i_vec + jnp.left_shift(jnp.right_shift(w, 4), 8)
                col = jnp.bitwise_and(w, 15)
                bword = plsc.load_gather(bwin_v, [row, col])
                oword = plsc.load_gather(owin_v, [row, col])
                e10 = (bword * bs + oword) * 10
                slot = _J * 16 * b + 16 * j
                q = jnp.right_shift(e10, 4)
                slot2 = jnp.left_shift(iota + slot, 1)
                plsc.store_scatter(eilist_v, [slot2], q)
                plsc.store_scatter(
                    eilist_v, [slot2 + 1],
                    jnp.minimum(q + 1, _N_ENTRIES * 10 // 16 - 1))
                lowlist_v[pl.ds(slot, 16)] = jnp.bitwise_and(e10, 15)
            return c

        lax.fori_loop(jnp.int32(0), jnp.int32(_NBH), stage3, 0)

        descs = []
        for c in range(2 * _SPH // _CH):
            descs.append(pltpu.async_copy(
                entries_hbm.at[eilist_v.at[pl.ds(_CH * c, _CH)]],
                rows_v.at[pl.ds(_CH * c, _CH)], sem_b))
        for d in descs:
            d.wait()

        def stage5(b, c, h=h):
            i0 = 128 * h + 16 * b
            sz = sizes_v[pl.ds(i0, 16)]
            accs = [jnp.zeros((16,), jnp.int32) for _ in range(10)]
            for j in range(_J):
                m = sz > j
                slot = _J * 16 * b + 16 * j
                low = lowlist_v[pl.ds(slot, 16)]
                fbase = jnp.left_shift(iota + slot, 5) + low
                for w in range(10):
                    fp = fbase + w
                    val = plsc.load_gather(
                        rows_v,
                        [jnp.right_shift(fp, 4), jnp.bitwise_and(fp, 15)])
                    accs[w] = jnp.bitwise_xor(accs[w], jnp.where(m, val, 0))
            obase = jnp.left_shift(iota + i0, 4)
            for w in range(10):
                plsc.store_scatter(outbuf_v, [obase + w], accs[w])
            return c

        lax.fori_loop(jnp.int32(0), jnp.int32(_NBH), stage5, 0)

    pltpu.sync_copy(outbuf_v, out_hbm.at[pl.ds(base * 16, _HPW * 16)])


def kernel(entries, subset_blocks, subset_offsets, subset_starts, subset_sizes, block_size):
    entries32 = lax.bitcast_convert_type(entries, jnp.int32).reshape(
        _N_ENTRIES * 10 // 16, 16)
    blocks2d = subset_blocks.astype(jnp.int32).reshape(_T // 16, 16)
    offs2d = subset_offsets.astype(jnp.int32).reshape(_T // 16, 16)
    starts32 = subset_starts.astype(jnp.int32)
    sizes32 = subset_sizes.astype(jnp.int32)
    bs_arr = jnp.full((16,), block_size, jnp.int32)

    mesh = plsc.VectorSubcoreMesh(
        core_axis_name="c", subcore_axis_name="s", num_cores=2, num_subcores=16)
    out32 = pl.kernel(
        _hint_kernel,
        out_type=jax.ShapeDtypeStruct((_H * 16,), jnp.int32),
        mesh=mesh,
        compiler_params=pltpu.CompilerParams(
            needs_layout_passes=False, use_tc_tiling_on_sc=False),
        scratch_types=[
            pltpu.VMEM((_HPW,), jnp.int32),
            pltpu.VMEM((_HPW,), jnp.int32),
            pltpu.VMEM((2 * _HPW,), jnp.int32),
            pltpu.VMEM((2 * _HPW, 16), jnp.int32),
            pltpu.VMEM((2 * _HPW, 16), jnp.int32),
            pltpu.VMEM((2 * _SPH,), jnp.int32),
            pltpu.VMEM((_SPH,), jnp.int32),
            pltpu.VMEM((2 * _SPH, 16), jnp.int32),
            pltpu.VMEM((_HPW * 16,), jnp.int32),
            pltpu.VMEM((16,), jnp.int32),
            pltpu.SemaphoreType.DMA,
            pltpu.SemaphoreType.DMA,
        ],
    )(entries32, blocks2d, offs2d, starts32, sizes32, bs_arr)

    out = lax.bitcast_convert_type(
        out32.reshape(_H, 16)[:, :10].reshape(_H, 5, 2), jnp.int64)
    return out


# speedup vs baseline: 4.0999x; 4.0999x over previous
import jax
import jax.numpy as jnp
from jax import lax
from jax.experimental import pallas as pl
from jax.experimental.pallas import tpu as pltpu
from jax.experimental.pallas import tpu_sc as plsc

_N_ENTRIES = 262144
_H = 8192
_T = 131072
_J = 15
_W = 32
_HPW = _H // _W
_NB = _HPW // 16
_EPW = _HPW * _J
_CH = 128


def _hint_kernel(entries_hbm, blk_hbm, off_hbm, starts_hbm, sizes_hbm, bs_hbm,
                 out_hbm, starts_v, sizes_v, rowidx_v, bwin_v, owin_v, eilist_v,
                 rows_v, outbuf_v, bs_v, sem_a, sem_b):
    wid = lax.axis_index("s") * 2 + lax.axis_index("c")
    base = wid * _HPW
    iota = lax.iota(jnp.int32, 16)

    pltpu.sync_copy(starts_hbm.at[pl.ds(base, _HPW)], starts_v)
    pltpu.sync_copy(sizes_hbm.at[pl.ds(base, _HPW)], sizes_v)
    pltpu.sync_copy(bs_hbm, bs_v)

    def stage1(b, c):
        s = starts_v[pl.ds(16 * b, 16)]
        r = jnp.right_shift(s, 4)
        rowidx_v[pl.ds(16 * b, 16)] = r
        rowidx_v[pl.ds(_HPW + 16 * b, 16)] = r + 1
        return c

    lax.fori_loop(jnp.int32(0), jnp.int32(_NB), stage1, 0)

    descs = []
    for c in range(2 * _HPW // _CH):
        descs.append(pltpu.async_copy(
            blk_hbm.at[rowidx_v.at[pl.ds(_CH * c, _CH)]],
            bwin_v.at[pl.ds(_CH * c, _CH)], sem_a))
        descs.append(pltpu.async_copy(
            off_hbm.at[rowidx_v.at[pl.ds(_CH * c, _CH)]],
            owin_v.at[pl.ds(_CH * c, _CH)], sem_a))
    for d in descs:
        d.wait()

    def stage3(b, c):
        s = starts_v[pl.ds(16 * b, 16)]
        w0 = jnp.bitwise_and(s, 15)
        i_vec = iota + 16 * b
        bs = bs_v[...]
        for j in range(_J):
            w = w0 + j
            row = i_vec + jnp.left_shift(jnp.right_shift(w, 4), 8)
            col = jnp.bitwise_and(w, 15)
            bword = plsc.load_gather(bwin_v, [row, col])
            oword = plsc.load_gather(owin_v, [row, col])
            eilist_v[pl.ds(_J * 16 * b + 16 * j, 16)] = bword * bs + oword
        return c

    lax.fori_loop(jnp.int32(0), jnp.int32(_NB), stage3, 0)

    descs = []
    for c in range(_EPW // _CH):
        descs.append(pltpu.async_copy(
            entries_hbm.at[eilist_v.at[pl.ds(_CH * c, _CH)]],
            rows_v.at[pl.ds(_CH * c, _CH)], sem_b))
    for d in descs:
        d.wait()

    def stage5(b, c):
        sz = sizes_v[pl.ds(16 * b, 16)]
        accs = [jnp.zeros((16,), jnp.int32) for _ in range(10)]
        for j in range(_J):
            m = sz > j
            rows = iota + (_J * 16 * b + 16 * j)
            for w in range(10):
                val = plsc.load_gather(rows_v, [rows, jnp.full((16,), w, jnp.int32)])
                accs[w] = jnp.bitwise_xor(accs[w], jnp.where(m, val, 0))
        i_vec = jnp.left_shift(iota + 16 * b, 4)
        for w in range(10):
            plsc.store_scatter(outbuf_v, [i_vec + w], accs[w])
        return c

    lax.fori_loop(jnp.int32(0), jnp.int32(_NB), stage5, 0)

    pltpu.sync_copy(outbuf_v, out_hbm.at[pl.ds(base * 16, _HPW * 16)])


def kernel(entries, subset_blocks, subset_offsets, subset_starts, subset_sizes, block_size):
    entries16 = lax.bitcast_convert_type(
        jnp.pad(entries, ((0, 0), (0, 3))), jnp.int32).reshape(_N_ENTRIES, 16)
    blocks2d = subset_blocks.astype(jnp.int32).reshape(_T // 16, 16)
    offs2d = subset_offsets.astype(jnp.int32).reshape(_T // 16, 16)
    starts32 = subset_starts.astype(jnp.int32)
    sizes32 = subset_sizes.astype(jnp.int32)
    bs_arr = jnp.full((16,), block_size, jnp.int32)

    mesh = plsc.VectorSubcoreMesh(
        core_axis_name="c", subcore_axis_name="s", num_cores=2, num_subcores=16)
    out32 = pl.kernel(
        _hint_kernel,
        out_type=jax.ShapeDtypeStruct((_H * 16,), jnp.int32),
        mesh=mesh,
        compiler_params=pltpu.CompilerParams(
            needs_layout_passes=False, use_tc_tiling_on_sc=False),
        scratch_types=[
            pltpu.VMEM((_HPW,), jnp.int32),
            pltpu.VMEM((_HPW,), jnp.int32),
            pltpu.VMEM((2 * _HPW,), jnp.int32),
            pltpu.VMEM((2 * _HPW, 16), jnp.int32),
            pltpu.VMEM((2 * _HPW, 16), jnp.int32),
            pltpu.VMEM((_EPW,), jnp.int32),
            pltpu.VMEM((_EPW, 16), jnp.int32),
            pltpu.VMEM((_HPW * 16,), jnp.int32),
            pltpu.VMEM((16,), jnp.int32),
            pltpu.SemaphoreType.DMA,
            pltpu.SemaphoreType.DMA,
        ],
    )(entries16, blocks2d, offs2d, starts32, sizes32, bs_arr)

    out = lax.bitcast_convert_type(
        out32.reshape(_H, 16)[:, :10].reshape(_H, 5, 2), jnp.int64)
    return out
